# baseline (speedup 1.0000x reference)
import functools

import jax
import jax.numpy as jnp
from jax import lax
from jax.experimental import pallas as pl
from jax.experimental.pallas import tpu as pltpu
from jax.experimental.pallas import tpu_sc as plsc

NUM_CHARACTERS = 1000000
EMBED_DIM = 16
BATCH = 16384
NUM_STAGES = 64

_NC = 2
_NS = 16
_NW = _NC * _NS
_B2 = 2 * BATCH
_BPW = _B2 // _NW
_CHUNK = 128
_STAGE = 256
_TROWS = NUM_CHARACTERS // 8


@functools.partial(
    pl.kernel,
    out_type=jax.ShapeDtypeStruct((_B2, 128), jnp.float32),
    mesh=plsc.VectorSubcoreMesh(
        core_axis_name="c", subcore_axis_name="s",
        num_cores=_NC, num_subcores=_NS),
    scratch_types=[
        pltpu.VMEM((_BPW,), jnp.int32),
        pltpu.VMEM((_BPW,), jnp.int32),
        pltpu.VMEM((_STAGE, 128), jnp.float32),
        pltpu.SemaphoreType.DMA,
    ],
    compiler_params=pltpu.CompilerParams(use_tc_tiling_on_sc=False),
)
def _sc_gather(idx_hbm, table_hbm, out_hbm, idx_v, row_v, rows_v, sem):
    wid = lax.axis_index("s") * _NC + lax.axis_index("c")
    base = wid * _BPW
    pltpu.sync_copy(idx_hbm.at[pl.ds(base, _BPW)], idx_v)
    for j in range(_BPW // 16):
        sl = pl.ds(j * 16, 16)
        row_v[sl] = jnp.right_shift(idx_v[sl], 3)
    for r in range(_BPW // _STAGE):
        copies = []
        for j in range(_STAGE // _CHUNK):
            src = pl.ds(r * _STAGE + j * _CHUNK, _CHUNK)
            dst = pl.ds(j * _CHUNK, _CHUNK)
            copies.append(pltpu.async_copy(
                table_hbm.at[row_v.at[src]], rows_v.at[dst], sem))
        for c in copies:
            c.wait()
        pltpu.sync_copy(rows_v, out_hbm.at[pl.ds(base + r * _STAGE, _STAGE)])


def _mlp_body(gw_ref, gl_ref, xw_ref, xl_ref,
              w1a_ref, w1b_ref, b1_ref, w2_ref, b2_ref, o_ref):
    lane = jax.lax.broadcasted_iota(jnp.int32, (_MLP_BLK, 128), 1) // 16
    mw = (lane == (xw_ref[...] & 7)).astype(jnp.float32)
    ml = (lane == (xl_ref[...] & 7)).astype(jnp.float32)
    z = jnp.dot(gw_ref[...] * mw, w1a_ref[...],
                preferred_element_type=jnp.float32)
    z = z + jnp.dot(gl_ref[...] * ml, w1b_ref[...],
                    preferred_element_type=jnp.float32)
    z = jnp.maximum(z + b1_ref[...], 0.0)
    o_ref[...] = (
        jnp.dot(z, w2_ref[...], preferred_element_type=jnp.float32)
        + b2_ref[...])


_MLP_BLK = 2048


def _mlp(gw, gl, xw, xl, W1a8, W1b8, b1, W2, b2):
    return pl.pallas_call(
        _mlp_body,
        grid=(BATCH // _MLP_BLK,),
        in_specs=[
            pl.BlockSpec((_MLP_BLK, 128), lambda i: (i, 0)),
            pl.BlockSpec((_MLP_BLK, 128), lambda i: (i, 0)),
            pl.BlockSpec((_MLP_BLK, 1), lambda i: (i, 0)),
            pl.BlockSpec((_MLP_BLK, 1), lambda i: (i, 0)),
            pl.BlockSpec((128, 64), lambda i: (0, 0)),
            pl.BlockSpec((128, 64), lambda i: (0, 0)),
            pl.BlockSpec((1, 64), lambda i: (0, 0)),
            pl.BlockSpec((64, NUM_STAGES), lambda i: (0, 0)),
            pl.BlockSpec((1, NUM_STAGES), lambda i: (0, 0)),
        ],
        out_specs=pl.BlockSpec((_MLP_BLK, NUM_STAGES), lambda i: (i, 0)),
        out_shape=jax.ShapeDtypeStruct((BATCH, NUM_STAGES), jnp.float32),
    )(gw, gl, xw, xl, W1a8, W1b8,
      b1.reshape(1, 64), W2, b2.reshape(1, NUM_STAGES))


def kernel(x, emb, W1, b1, W2, b2):
    x = x.astype(jnp.int32)
    idx = jnp.concatenate([x[:, 0], x[:, 1]])
    table = emb.reshape(_TROWS, 128)
    g = _sc_gather(idx, table)
    W1a8 = jnp.tile(W1[:EMBED_DIM], (8, 1))
    W1b8 = jnp.tile(W1[EMBED_DIM:], (8, 1))
    return _mlp(g[:BATCH], g[BATCH:], x[:, 0:1], x[:, 1:2],
                W1a8, W1b8, b1, W2, b2)


# speedup vs baseline: 2.1614x; 2.1614x over previous
import functools

import jax
import jax.numpy as jnp
from jax import lax
from jax.experimental import pallas as pl
from jax.experimental.pallas import tpu as pltpu
from jax.experimental.pallas import tpu_sc as plsc

NUM_CHARACTERS = 1000000
EMBED_DIM = 16
BATCH = 16384
NUM_STAGES = 64

_TW = 8192
_TGRID = -(-NUM_CHARACTERS // _TW)
_TBLROWS = _TGRID * _TW // 8


def _transpose_body(i_ref, eye_ref, o_ref):
    z = jnp.zeros((_TW // 8, 128), jnp.float32)
    for h in range(8):
        z = z + jax.lax.dot_general(
            i_ref[:, h * (_TW // 8):(h + 1) * (_TW // 8)],
            eye_ref[h * EMBED_DIM:(h + 1) * EMBED_DIM, :],
            (((0,), (0,)), ((), ())),
            preferred_element_type=jnp.float32)
    o_ref[...] = z


def _transpose(embT, eye):
    return pl.pallas_call(
        _transpose_body,
        grid=(_TGRID,),
        in_specs=[
            pl.BlockSpec((EMBED_DIM, _TW), lambda i: (0, i)),
            pl.BlockSpec((128, 128), lambda i: (0, 0)),
        ],
        out_specs=pl.BlockSpec((_TW // 8, 128), lambda i: (i, 0)),
        out_shape=jax.ShapeDtypeStruct((_TBLROWS, 128), jnp.float32),
    )(embT, eye)


_NC = 2
_NS = 16
_NW = _NC * _NS
_B2 = 2 * BATCH
_BPW = _B2 // _NW
_CHUNK = 128
_STAGE = 256
_TROWS = NUM_CHARACTERS // 8


@functools.partial(
    pl.kernel,
    out_type=jax.ShapeDtypeStruct((_B2, 128), jnp.float32),
    mesh=plsc.VectorSubcoreMesh(
        core_axis_name="c", subcore_axis_name="s",
        num_cores=_NC, num_subcores=_NS),
    scratch_types=[
        pltpu.VMEM((_BPW,), jnp.int32),
        pltpu.VMEM((_STAGE, 128), jnp.float32),
        pltpu.SemaphoreType.DMA,
    ],
)
def _sc_gather(row_hbm, table_hbm, out_hbm, idx_v, rows_v, sem):
    wid = lax.axis_index("s") * _NC + lax.axis_index("c")
    base = wid * _BPW
    pltpu.sync_copy(row_hbm.at[pl.ds(base, _BPW)], idx_v)
    for r in range(_BPW // _STAGE):
        copies = []
        for j in range(_STAGE // _CHUNK):
            src = pl.ds(r * _STAGE + j * _CHUNK, _CHUNK)
            dst = pl.ds(j * _CHUNK, _CHUNK)
            copies.append(pltpu.async_copy(
                table_hbm.at[idx_v.at[src]], rows_v.at[dst], sem))
        for c in copies:
            c.wait()
        pltpu.sync_copy(rows_v, out_hbm.at[pl.ds(base + r * _STAGE, _STAGE)])


def _mlp_body(gw_ref, gl_ref, xw_ref, xl_ref,
              w1a_ref, w1b_ref, b1_ref, w2_ref, b2_ref, o_ref):
    lane = jax.lax.broadcasted_iota(jnp.int32, (_MLP_BLK, 128), 1) // 16
    mw = (lane == ((xw_ref[...] >> 10) & 7)).astype(jnp.float32)
    ml = (lane == ((xl_ref[...] >> 10) & 7)).astype(jnp.float32)
    z = jnp.dot(gw_ref[...] * mw, w1a_ref[...],
                preferred_element_type=jnp.float32)
    z = z + jnp.dot(gl_ref[...] * ml, w1b_ref[...],
                    preferred_element_type=jnp.float32)
    z = jnp.maximum(z + b1_ref[...], 0.0)
    o_ref[...] = (
        jnp.dot(z, w2_ref[...], preferred_element_type=jnp.float32)
        + b2_ref[...])


_MLP_BLK = 2048


def _mlp(gw, gl, xw, xl, W1a8, W1b8, b1, W2, b2):
    return pl.pallas_call(
        _mlp_body,
        grid=(BATCH // _MLP_BLK,),
        in_specs=[
            pl.BlockSpec((_MLP_BLK, 128), lambda i: (i, 0)),
            pl.BlockSpec((_MLP_BLK, 128), lambda i: (i, 0)),
            pl.BlockSpec((_MLP_BLK, 1), lambda i: (i, 0)),
            pl.BlockSpec((_MLP_BLK, 1), lambda i: (i, 0)),
            pl.BlockSpec((128, 64), lambda i: (0, 0)),
            pl.BlockSpec((128, 64), lambda i: (0, 0)),
            pl.BlockSpec((1, 64), lambda i: (0, 0)),
            pl.BlockSpec((64, NUM_STAGES), lambda i: (0, 0)),
            pl.BlockSpec((1, NUM_STAGES), lambda i: (0, 0)),
        ],
        out_specs=pl.BlockSpec((_MLP_BLK, NUM_STAGES), lambda i: (i, 0)),
        out_shape=jax.ShapeDtypeStruct((BATCH, NUM_STAGES), jnp.float32),
    )(gw, gl, xw, xl, W1a8, W1b8,
      b1.reshape(1, 64), W2, b2.reshape(1, NUM_STAGES))


def kernel(x, emb, W1, b1, W2, b2):
    x = x.astype(jnp.int32)
    idx = jnp.concatenate([x[:, 0], x[:, 1]])
    rows = ((idx >> 13) << 10) + (idx & 1023)
    table = _transpose(emb.T, jnp.eye(128, dtype=jnp.float32))
    g = _sc_gather(rows, table)
    W1a8 = jnp.tile(W1[:EMBED_DIM], (8, 1))
    W1b8 = jnp.tile(W1[EMBED_DIM:], (8, 1))
    return _mlp(g[:BATCH], g[BATCH:], x[:, 0:1], x[:, 1:2],
                W1a8, W1b8, b1, W2, b2)


# speedup vs baseline: 3.6501x; 1.6887x over previous
import functools

import jax
import jax.numpy as jnp
from jax import lax
from jax.experimental import pallas as pl
from jax.experimental.pallas import tpu as pltpu
from jax.experimental.pallas import tpu_sc as plsc

NUM_CHARACTERS = 1000000
EMBED_DIM = 16
BATCH = 16384
NUM_STAGES = 64

_TW = 65536
_TGRID = -(-NUM_CHARACTERS // _TW)
_TM = _TW // 8
_TBLROWS = _TGRID * _TM


def _transpose_body(i_ref, eye_ref, o_ref):
    z = jnp.zeros((_TM, 128), jnp.float32)
    x_bf = i_ref[...].astype(jnp.bfloat16)
    e_bf = eye_ref[...].astype(jnp.bfloat16)
    for h in range(8):
        z = z + jax.lax.dot_general(
            x_bf[:, h * _TM:(h + 1) * _TM],
            e_bf[h * EMBED_DIM:(h + 1) * EMBED_DIM, :],
            (((0,), (0,)), ((), ())),
            preferred_element_type=jnp.float32)
    o_ref[...] = z


def _transpose(embT, eye):
    return pl.pallas_call(
        _transpose_body,
        grid=(_TGRID,),
        in_specs=[
            pl.BlockSpec((EMBED_DIM, _TW), lambda i: (0, i)),
            pl.BlockSpec((128, 128), lambda i: (0, 0)),
        ],
        out_specs=pl.BlockSpec((_TM, 128), lambda i: (i, 0)),
        out_shape=jax.ShapeDtypeStruct((_TBLROWS, 128), jnp.float32),
    )(embT, eye)


_NC = 2
_NS = 16
_NW = _NC * _NS
_B2 = 2 * BATCH
_BPW = _B2 // _NW
_CHUNK = 128
_STAGE = 256


@functools.partial(
    pl.kernel,
    out_type=jax.ShapeDtypeStruct((_B2, 128), jnp.float32),
    mesh=plsc.VectorSubcoreMesh(
        core_axis_name="c", subcore_axis_name="s",
        num_cores=_NC, num_subcores=_NS),
    scratch_types=[
        pltpu.VMEM((_BPW,), jnp.int32),
        pltpu.VMEM((2, _STAGE, 128), jnp.float32),
        pltpu.SemaphoreType.DMA,
        pltpu.SemaphoreType.DMA,
    ],
)
def _sc_gather(row_hbm, table_hbm, out_hbm, idx_v, rows_v, sem, wsem):
    wid = lax.axis_index("s") * _NC + lax.axis_index("c")
    base = wid * _BPW
    pltpu.sync_copy(row_hbm.at[pl.ds(base, _BPW)], idx_v)
    nround = _BPW // _STAGE
    gathers, writes = {}, {}

    def fire(r):
        buf = rows_v.at[r % 2]
        return [pltpu.async_copy(
            table_hbm.at[idx_v.at[pl.ds(r * _STAGE + j * _CHUNK, _CHUNK)]],
            buf.at[pl.ds(j * _CHUNK, _CHUNK)], sem)
            for j in range(_STAGE // _CHUNK)]

    def write_out(r):
        for c in gathers[r]:
            c.wait()
        writes[r] = pltpu.async_copy(
            rows_v.at[r % 2],
            out_hbm.at[pl.ds(base + r * _STAGE, _STAGE)], wsem)

    for r in range(nround):
        if r - 2 in writes:
            writes[r - 2].wait()
        gathers[r] = fire(r)
        if r - 1 in gathers:
            write_out(r - 1)
    write_out(nround - 1)
    writes[nround - 2].wait()
    writes[nround - 1].wait()


def _mlp_body(gw_ref, gl_ref, xw_ref, xl_ref,
              w1a_ref, w1b_ref, b1_ref, w2_ref, b2_ref, o_ref):
    lane = jax.lax.broadcasted_iota(jnp.int32, (_MLP_BLK, 128), 1) // 16
    mw = (lane == ((xw_ref[...] >> 13) & 7)).astype(jnp.float32)
    ml = (lane == ((xl_ref[...] >> 13) & 7)).astype(jnp.float32)
    z = jnp.dot(gw_ref[...] * mw, w1a_ref[...],
                preferred_element_type=jnp.float32)
    z = z + jnp.dot(gl_ref[...] * ml, w1b_ref[...],
                    preferred_element_type=jnp.float32)
    z = jnp.maximum(z + b1_ref[...], 0.0)
    o_ref[...] = (
        jnp.dot(z, w2_ref[...], preferred_element_type=jnp.float32)
        + b2_ref[...])


_MLP_BLK = 2048


def _mlp(g, xw, xl, W1a8, W1b8, b1, W2, b2):
    nblk = BATCH // _MLP_BLK
    return pl.pallas_call(
        _mlp_body,
        grid=(nblk,),
        in_specs=[
            pl.BlockSpec((_MLP_BLK, 128), lambda i: (i, 0)),
            pl.BlockSpec((_MLP_BLK, 128), lambda i: (i + nblk, 0)),
            pl.BlockSpec((_MLP_BLK, 1), lambda i: (i, 0)),
            pl.BlockSpec((_MLP_BLK, 1), lambda i: (i, 0)),
            pl.BlockSpec((128, 64), lambda i: (0, 0)),
            pl.BlockSpec((128, 64), lambda i: (0, 0)),
            pl.BlockSpec((1, 64), lambda i: (0, 0)),
            pl.BlockSpec((64, NUM_STAGES), lambda i: (0, 0)),
            pl.BlockSpec((1, NUM_STAGES), lambda i: (0, 0)),
        ],
        out_specs=pl.BlockSpec((_MLP_BLK, NUM_STAGES), lambda i: (i, 0)),
        out_shape=jax.ShapeDtypeStruct((BATCH, NUM_STAGES), jnp.float32),
    )(g, g, xw, xl, W1a8, W1b8,
      b1.reshape(1, 64), W2, b2.reshape(1, NUM_STAGES))


def kernel(x, emb, W1, b1, W2, b2):
    x = x.astype(jnp.int32)
    idx = jnp.concatenate([x[:, 0], x[:, 1]])
    rows = ((idx >> 16) << 13) + (idx & 8191)
    table = _transpose(emb.T, jnp.eye(128, dtype=jnp.float32))
    g = _sc_gather(rows, table)
    W1a8 = jnp.tile(W1[:EMBED_DIM], (8, 1))
    W1b8 = jnp.tile(W1[EMBED_DIM:], (8, 1))
    return _mlp(g, x[:, 0:1], x[:, 1:2],
                W1a8, W1b8, b1, W2, b2)


# speedup vs baseline: 3.7689x; 1.0325x over previous
import functools

import jax
import jax.numpy as jnp
from jax import lax
from jax.experimental import pallas as pl
from jax.experimental.pallas import tpu as pltpu
from jax.experimental.pallas import tpu_sc as plsc

NUM_CHARACTERS = 1000000
EMBED_DIM = 16
BATCH = 16384
NUM_STAGES = 64

_TW = 65536
_TGRID = -(-NUM_CHARACTERS // _TW)
_TM = _TW // 8
_TBLROWS = _TGRID * _TM


def _transpose_body(i_ref, eye_ref, o_ref):
    z = jnp.zeros((_TM, 128), jnp.float32)
    x_bf = i_ref[...].astype(jnp.bfloat16)
    e_bf = eye_ref[...].astype(jnp.bfloat16)
    for h in range(8):
        z = z + jax.lax.dot_general(
            x_bf[:, h * _TM:(h + 1) * _TM],
            e_bf[h * EMBED_DIM:(h + 1) * EMBED_DIM, :],
            (((0,), (0,)), ((), ())),
            preferred_element_type=jnp.float32)
    o_ref[...] = z


def _transpose(embT, eye):
    return pl.pallas_call(
        _transpose_body,
        grid=(_TGRID,),
        in_specs=[
            pl.BlockSpec((EMBED_DIM, _TW), lambda i: (0, i)),
            pl.BlockSpec((128, 128), lambda i: (0, 0)),
        ],
        out_specs=pl.BlockSpec((_TM, 128), lambda i: (i, 0)),
        out_shape=jax.ShapeDtypeStruct((_TBLROWS, 128), jnp.float32),
    )(embT, eye)


_NC = 2
_NS = 16
_NW = _NC * _NS
_B2 = 2 * BATCH
_BPW = _B2 // _NW
_CHUNK = 128
_STAGE = 256
_ROUNDS = _BPW // _STAGE


@functools.partial(
    pl.kernel,
    out_type=jax.ShapeDtypeStruct((_B2 * EMBED_DIM,), jnp.float32),
    mesh=plsc.VectorSubcoreMesh(
        core_axis_name="c", subcore_axis_name="s",
        num_cores=_NC, num_subcores=_NS),
    scratch_types=[
        pltpu.VMEM((_BPW,), jnp.int32),
        pltpu.VMEM((_BPW,), jnp.int32),
        pltpu.VMEM((_STAGE, 128), jnp.float32),
        pltpu.VMEM((_STAGE, 128), jnp.float32),
        pltpu.VMEM((_STAGE * EMBED_DIM,), jnp.float32),
        pltpu.VMEM((_STAGE * EMBED_DIM,), jnp.float32),
        pltpu.SemaphoreType.DMA,
        pltpu.SemaphoreType.DMA,
    ],
    compiler_params=pltpu.CompilerParams(needs_layout_passes=False),
)
def _sc_gather(pk_hbm, table_hbm, out_hbm, idx_v, row_v,
               rows_a, rows_b, ext_a, ext_b, sem, wsem):
    wid = lax.axis_index("s") * _NC + lax.axis_index("c")
    base = wid * _BPW
    pltpu.sync_copy(pk_hbm.at[pl.ds(base, _BPW)], idx_v)
    for j in range(_BPW // 16):
        sl = pl.ds(j * 16, 16)
        row_v[sl] = jnp.bitwise_and(idx_v[sl], 0x1FFFF)
    gathers, writes = {}, {}
    lane16 = lax.iota(jnp.int32, 16)

    def fire(r):
        buf = rows_a if r % 2 == 0 else rows_b
        return [pltpu.async_copy(
            table_hbm.at[row_v.at[pl.ds(r * _STAGE + j * _CHUNK, _CHUNK)]],
            buf.at[pl.ds(j * _CHUNK, _CHUNK)], sem)
            for j in range(_STAGE // _CHUNK)]

    def extract_and_write(r):
        for c in gathers[r]:
            c.wait()
        buf = rows_a if r % 2 == 0 else rows_b
        ext = ext_a if r % 2 == 0 else ext_b

        def body(i, _):
            pv = plsc.load_gather(
                idx_v, [jnp.full((16,), r * _STAGE + i, jnp.int32)])
            cols = jnp.right_shift(pv, 17) * 16 + lane16
            v = plsc.load_gather(buf, [jnp.full((16,), i, jnp.int32), cols])
            ext[pl.ds(i * EMBED_DIM, EMBED_DIM)] = v
            return 0

        lax.fori_loop(0, _STAGE, body, 0)
        writes[r] = pltpu.async_copy(
            ext,
            out_hbm.at[pl.ds((base + r * _STAGE) * EMBED_DIM,
                             _STAGE * EMBED_DIM)], wsem)

    for r in range(_ROUNDS):
        if r - 2 in writes:
            writes[r - 2].wait()
        gathers[r] = fire(r)
        if r - 1 in gathers:
            extract_and_write(r - 1)
    extract_and_write(_ROUNDS - 1)
    writes[_ROUNDS - 2].wait()
    writes[_ROUNDS - 1].wait()


def _mlp_body(w8_ref, l8_ref, w1a_ref, w1b_ref, b1_ref, w2_ref, b2_ref,
              o_ref):
    z = jnp.dot(w8_ref[...], w1a_ref[...], preferred_element_type=jnp.float32)
    z = z + jnp.dot(l8_ref[...], w1b_ref[...],
                    preferred_element_type=jnp.float32)
    z = jnp.maximum(z + b1_ref[...], 0.0)
    o_ref[...] = (
        jnp.dot(z, w2_ref[...], preferred_element_type=jnp.float32)
        + b2_ref[...])


_MLP_BLK = 512
_G8ROWS = _B2 * EMBED_DIM // 128


def _mlp(g8, W1a, W1b, b1x, W2x, b2x):
    nblk = _G8ROWS // 2 // _MLP_BLK
    return pl.pallas_call(
        _mlp_body,
        grid=(nblk,),
        in_specs=[
            pl.BlockSpec((_MLP_BLK, 128), lambda i: (i, 0)),
            pl.BlockSpec((_MLP_BLK, 128), lambda i: (i + nblk, 0)),
            pl.BlockSpec((128, 512), lambda i: (0, 0)),
            pl.BlockSpec((128, 512), lambda i: (0, 0)),
            pl.BlockSpec((1, 512), lambda i: (0, 0)),
            pl.BlockSpec((512, 512), lambda i: (0, 0)),
            pl.BlockSpec((1, 512), lambda i: (0, 0)),
        ],
        out_specs=pl.BlockSpec((_MLP_BLK, 512), lambda i: (i, 0)),
        out_shape=jax.ShapeDtypeStruct((_G8ROWS // 2, 512), jnp.float32),
    )(g8, g8, W1a, W1b, b1x, W2x, b2x)


def _blockdiag(w, n=8):
    a, b = w.shape
    out = jnp.zeros((n, a, n, b), w.dtype)
    out = out.at[jnp.arange(n), :, jnp.arange(n), :].set(w)
    return out.reshape(n * a, n * b)


def kernel(x, emb, W1, b1, W2, b2):
    x = x.astype(jnp.int32)
    idx = jnp.concatenate([x[:, 0], x[:, 1]])
    rows = ((idx >> 16) << 13) + (idx & 8191)
    lane_grp = (idx >> 13) & 7
    packed = (lane_grp << 17) | rows
    table = _transpose(emb.T, jnp.eye(128, dtype=jnp.float32))
    g = _sc_gather(packed, table)
    g8 = g.reshape(_G8ROWS, 128)
    out8 = _mlp(g8,
                _blockdiag(W1[:EMBED_DIM]), _blockdiag(W1[EMBED_DIM:]),
                jnp.tile(b1, 8).reshape(1, 512),
                _blockdiag(W2),
                jnp.tile(b2, 8).reshape(1, 512))
    return out8.reshape(BATCH, NUM_STAGES)


# speedup vs baseline: 4.0030x; 1.0621x over previous
import functools

import jax
import jax.numpy as jnp
from jax import lax
from jax.experimental import pallas as pl
from jax.experimental.pallas import tpu as pltpu
from jax.experimental.pallas import tpu_sc as plsc

NUM_CHARACTERS = 1000000
EMBED_DIM = 16
BATCH = 16384
NUM_STAGES = 64

_TW = 65536
_TGRID = -(-NUM_CHARACTERS // _TW)
_TM = _TW // 8
_TBLROWS = _TGRID * _TM


def _transpose_body(i_ref, eye_ref, o_ref):
    z = jnp.zeros((_TM, 128), jnp.float32)
    x_bf = i_ref[...].astype(jnp.bfloat16)
    e_bf = eye_ref[...].astype(jnp.bfloat16)
    for h in range(8):
        z = z + jax.lax.dot_general(
            x_bf[:, h * _TM:(h + 1) * _TM],
            e_bf[h * EMBED_DIM:(h + 1) * EMBED_DIM, :],
            (((0,), (0,)), ((), ())),
            preferred_element_type=jnp.float32)
    o_ref[...] = z


def _transpose(embT, eye):
    return pl.pallas_call(
        _transpose_body,
        grid=(_TGRID,),
        in_specs=[
            pl.BlockSpec((EMBED_DIM, _TW), lambda i: (0, i)),
            pl.BlockSpec((128, 128), lambda i: (0, 0)),
        ],
        out_specs=pl.BlockSpec((_TM, 128), lambda i: (i, 0)),
        out_shape=jax.ShapeDtypeStruct((_TBLROWS, 128), jnp.float32),
    )(embT, eye)


_NC = 2
_NS = 16
_NW = _NC * _NS
_B2 = 2 * BATCH
_BPW = _B2 // _NW
_CHUNK = 128
_STAGE = 256
_ROUNDS = _BPW // _STAGE


@functools.partial(
    pl.kernel,
    out_type=jax.ShapeDtypeStruct((2 * EMBED_DIM, BATCH), jnp.float32),
    mesh=plsc.VectorSubcoreMesh(
        core_axis_name="c", subcore_axis_name="s",
        num_cores=_NC, num_subcores=_NS),
    scratch_types=[
        pltpu.VMEM((_BPW,), jnp.int32),
        pltpu.VMEM((_BPW,), jnp.int32),
        pltpu.VMEM((_STAGE, 128), jnp.float32),
        pltpu.VMEM((_STAGE, 128), jnp.float32),
        pltpu.VMEM((EMBED_DIM, _STAGE), jnp.float32),
        pltpu.VMEM((EMBED_DIM, _STAGE), jnp.float32),
        pltpu.SemaphoreType.DMA,
        pltpu.SemaphoreType.DMA,
    ],
    compiler_params=pltpu.CompilerParams(needs_layout_passes=False),
)
def _sc_gather(pk_hbm, table_hbm, out_hbm, idx_v, row_v,
               rows_a, rows_b, ext_a, ext_b, sem, wsem):
    wid = lax.axis_index("s") * _NC + lax.axis_index("c")
    base = wid * _BPW
    dbase = EMBED_DIM * (wid // 16)
    cbase = (wid % 16) * _BPW
    pltpu.sync_copy(pk_hbm.at[pl.ds(base, _BPW)], idx_v)
    for j in range(_BPW // 16):
        sl = pl.ds(j * 16, 16)
        row_v[sl] = jnp.bitwise_and(idx_v[sl], 0x1FFFF)
    gathers, writes = {}, {}
    lane16 = lax.iota(jnp.int32, 16)

    def fire(r):
        buf = rows_a if r % 2 == 0 else rows_b
        return [pltpu.async_copy(
            table_hbm.at[row_v.at[pl.ds(r * _STAGE + j * _CHUNK, _CHUNK)]],
            buf.at[pl.ds(j * _CHUNK, _CHUNK)], sem)
            for j in range(_STAGE // _CHUNK)]

    def extract_and_write(r):
        for c in gathers[r]:
            c.wait()
        buf = rows_a if r % 2 == 0 else rows_b
        ext = ext_a if r % 2 == 0 else ext_b

        def body(i, _):
            pv = plsc.load_gather(
                idx_v, [jnp.full((16,), r * _STAGE + i, jnp.int32)])
            cols = jnp.right_shift(pv, 17) * 16 + lane16
            v = plsc.load_gather(buf, [jnp.full((16,), i, jnp.int32), cols])
            plsc.store_scatter(ext, [lane16, jnp.full((16,), i, jnp.int32)], v)
            return 0

        lax.fori_loop(0, _STAGE, body, 0)
        writes[r] = pltpu.async_copy(
            ext,
            out_hbm.at[pl.ds(dbase, EMBED_DIM),
                       pl.ds(cbase + r * _STAGE, _STAGE)], wsem)

    for r in range(_ROUNDS):
        if r - 2 in writes:
            writes[r - 2].wait()
        gathers[r] = fire(r)
        if r - 1 in gathers:
            extract_and_write(r - 1)
    extract_and_write(_ROUNDS - 1)
    writes[_ROUNDS - 2].wait()
    writes[_ROUNDS - 1].wait()


def _mlp_body(g_ref, w1_ref, b1_ref, w2_ref, b2_ref, o_ref):
    z = jax.lax.dot_general(w1_ref[...], g_ref[...], (((0,), (0,)), ((), ())),
                            preferred_element_type=jnp.float32)
    z = jnp.maximum(z + b1_ref[...], 0.0)
    o_ref[...] = (
        jax.lax.dot_general(w2_ref[...], z, (((0,), (0,)), ((), ())),
                            preferred_element_type=jnp.float32)
        + b2_ref[...])


_MLP_BLK = 2048


def _mlp(g, W1, b1, W2, b2):
    return pl.pallas_call(
        _mlp_body,
        grid=(BATCH // _MLP_BLK,),
        in_specs=[
            pl.BlockSpec((2 * EMBED_DIM, _MLP_BLK), lambda i: (0, i)),
            pl.BlockSpec((2 * EMBED_DIM, 64), lambda i: (0, 0)),
            pl.BlockSpec((64, 1), lambda i: (0, 0)),
            pl.BlockSpec((64, NUM_STAGES), lambda i: (0, 0)),
            pl.BlockSpec((NUM_STAGES, 1), lambda i: (0, 0)),
        ],
        out_specs=pl.BlockSpec((NUM_STAGES, _MLP_BLK), lambda i: (0, i)),
        out_shape=jax.ShapeDtypeStruct((NUM_STAGES, BATCH), jnp.float32),
    )(g, W1, b1.reshape(64, 1), W2, b2.reshape(NUM_STAGES, 1))


def kernel(x, emb, W1, b1, W2, b2):
    x = x.astype(jnp.int32)
    idx = jnp.concatenate([x[:, 0], x[:, 1]])
    rows = ((idx >> 16) << 13) + (idx & 8191)
    lane_grp = (idx >> 13) & 7
    packed = (lane_grp << 17) | rows
    table = _transpose(emb.T, jnp.eye(128, dtype=jnp.float32))
    g = _sc_gather(packed, table)
    out_t = _mlp(g, W1, b1, W2, b2)
    return out_t.T


# speedup vs baseline: 5.5325x; 1.3821x over previous
import functools

import jax
import jax.numpy as jnp
from jax import lax
from jax.experimental import pallas as pl
from jax.experimental.pallas import tpu as pltpu
from jax.experimental.pallas import tpu_sc as plsc

NUM_CHARACTERS = 1000000
EMBED_DIM = 16
BATCH = 16384
NUM_STAGES = 64

_TW = 65536
_TGRID = -(-NUM_CHARACTERS // _TW)
_TM = _TW // 8
_TBLROWS = _TGRID * _TM


def _transpose_body(i_ref, eye_ref, o_ref):
    x = i_ref[...]
    stacked = jnp.concatenate(
        [x[:, h * _TM:(h + 1) * _TM] for h in range(8)], axis=0)
    o_ref[...] = jax.lax.dot_general(
        stacked, eye_ref[...], (((0,), (0,)), ((), ())),
        preferred_element_type=jnp.float32)


def _transpose(embT, eye):
    return pl.pallas_call(
        _transpose_body,
        grid=(_TGRID,),
        in_specs=[
            pl.BlockSpec((EMBED_DIM, _TW), lambda i: (0, i)),
            pl.BlockSpec((128, 128), lambda i: (0, 0)),
        ],
        out_specs=pl.BlockSpec((_TM, 128), lambda i: (i, 0)),
        out_shape=jax.ShapeDtypeStruct((_TBLROWS, 128), jnp.float32),
    )(embT, eye)


_NC = 2
_NS = 16
_NW = _NC * _NS
_B2 = 2 * BATCH
_BPW = _B2 // _NW
_CHUNK = 128
_STAGE = 256
_ROUNDS = _BPW // _STAGE


@functools.partial(
    pl.kernel,
    out_type=jax.ShapeDtypeStruct((2 * EMBED_DIM, BATCH), jnp.float32),
    mesh=plsc.VectorSubcoreMesh(
        core_axis_name="c", subcore_axis_name="s",
        num_cores=_NC, num_subcores=_NS),
    scratch_types=[
        pltpu.VMEM((_BPW,), jnp.int32),
        pltpu.VMEM((_BPW,), jnp.int32),
        pltpu.VMEM((_STAGE, 128), jnp.float32),
        pltpu.VMEM((_STAGE, 128), jnp.float32),
        pltpu.VMEM((EMBED_DIM, _STAGE), jnp.float32),
        pltpu.VMEM((EMBED_DIM, _STAGE), jnp.float32),
        pltpu.SemaphoreType.DMA,
        pltpu.SemaphoreType.DMA,
    ],
    compiler_params=pltpu.CompilerParams(needs_layout_passes=False),
)
def _sc_gather(pk_hbm, table_hbm, out_hbm, idx_v, row_v,
               rows_a, rows_b, ext_a, ext_b, sem, wsem):
    wid = lax.axis_index("s") * _NC + lax.axis_index("c")
    base = wid * _BPW
    dbase = EMBED_DIM * (wid // 16)
    cbase = (wid % 16) * _BPW
    pltpu.sync_copy(pk_hbm.at[pl.ds(base, _BPW)], idx_v)
    for j in range(_BPW // 16):
        sl = pl.ds(j * 16, 16)
        row_v[sl] = jnp.bitwise_and(idx_v[sl], 0x1FFFF)
    gathers, writes = {}, {}
    lane16 = lax.iota(jnp.int32, 16)

    def fire(r):
        buf = rows_a if r % 2 == 0 else rows_b
        return [pltpu.async_copy(
            table_hbm.at[row_v.at[pl.ds(r * _STAGE + j * _CHUNK, _CHUNK)]],
            buf.at[pl.ds(j * _CHUNK, _CHUNK)], sem)
            for j in range(_STAGE // _CHUNK)]

    def extract_and_write(r):
        for c in gathers[r]:
            c.wait()
        buf = rows_a if r % 2 == 0 else rows_b
        ext = ext_a if r % 2 == 0 else ext_b

        def body(i, _):
            pv = plsc.load_gather(
                idx_v, [jnp.full((16,), r * _STAGE + i, jnp.int32)])
            cols = jnp.right_shift(pv, 17) * 16 + lane16
            v = plsc.load_gather(buf, [jnp.full((16,), i, jnp.int32), cols])
            plsc.store_scatter(ext, [lane16, jnp.full((16,), i, jnp.int32)], v)
            return 0

        lax.fori_loop(0, _STAGE, body, 0)
        writes[r] = pltpu.async_copy(
            ext,
            out_hbm.at[pl.ds(dbase, EMBED_DIM),
                       pl.ds(cbase + r * _STAGE, _STAGE)], wsem)

    for r in range(_ROUNDS):
        if r - 2 in writes:
            writes[r - 2].wait()
        gathers[r] = fire(r)
        if r - 1 in gathers:
            extract_and_write(r - 1)
    extract_and_write(_ROUNDS - 1)
    writes[_ROUNDS - 2].wait()
    writes[_ROUNDS - 1].wait()


def _mlp_body(g_ref, w1_ref, b1_ref, w2_ref, b2_ref, o_ref):
    z = jax.lax.dot_general(w1_ref[...], g_ref[...], (((0,), (0,)), ((), ())),
                            preferred_element_type=jnp.float32)
    z = jnp.maximum(z + b1_ref[...], 0.0)
    o_ref[...] = (
        jax.lax.dot_general(w2_ref[...], z, (((0,), (0,)), ((), ())),
                            preferred_element_type=jnp.float32)
        + b2_ref[...])


_MLP_BLK = 2048


def _mlp(g, W1, b1, W2, b2):
    return pl.pallas_call(
        _mlp_body,
        grid=(BATCH // _MLP_BLK,),
        in_specs=[
            pl.BlockSpec((2 * EMBED_DIM, _MLP_BLK), lambda i: (0, i)),
            pl.BlockSpec((2 * EMBED_DIM, 64), lambda i: (0, 0)),
            pl.BlockSpec((64, 1), lambda i: (0, 0)),
            pl.BlockSpec((64, NUM_STAGES), lambda i: (0, 0)),
            pl.BlockSpec((NUM_STAGES, 1), lambda i: (0, 0)),
        ],
        out_specs=pl.BlockSpec((NUM_STAGES, _MLP_BLK), lambda i: (0, i)),
        out_shape=jax.ShapeDtypeStruct((NUM_STAGES, BATCH), jnp.float32),
    )(g, W1, b1.reshape(64, 1), W2, b2.reshape(NUM_STAGES, 1))


def kernel(x, emb, W1, b1, W2, b2):
    x = x.astype(jnp.int32)
    idx = jnp.concatenate([x[:, 0], x[:, 1]])
    rows = ((idx >> 16) << 13) + (idx & 8191)
    lane_grp = (idx >> 13) & 7
    packed = (lane_grp << 17) | rows
    table = _transpose(emb.T, jnp.eye(128, dtype=jnp.float32))
    g = _sc_gather(packed, table)
    out_t = _mlp(g, W1, b1, W2, b2)
    return out_t.T


# speedup vs baseline: 6.1174x; 1.1057x over previous
import functools

import jax
import jax.numpy as jnp
from jax import lax
from jax.experimental import pallas as pl
from jax.experimental.pallas import tpu as pltpu
from jax.experimental.pallas import tpu_sc as plsc

NUM_CHARACTERS = 1000000
EMBED_DIM = 16
BATCH = 16384
NUM_STAGES = 64

_TW = 65536
_TGRID = -(-NUM_CHARACTERS // _TW)
_TM = _TW // 8
_TBLROWS = _TGRID * _TM


def _transpose_body(i_ref, eye_ref, o_ref):
    x = i_ref[...]
    stacked = jnp.concatenate(
        [x[:, h * _TM:(h + 1) * _TM] for h in range(8)], axis=0)
    o_ref[...] = jax.lax.dot_general(
        stacked, eye_ref[...], (((0,), (0,)), ((), ())),
        preferred_element_type=jnp.float32)


def _transpose(embT, eye):
    return pl.pallas_call(
        _transpose_body,
        grid=(_TGRID,),
        in_specs=[
            pl.BlockSpec((EMBED_DIM, _TW), lambda i: (0, i)),
            pl.BlockSpec((128, 128), lambda i: (0, 0)),
        ],
        out_specs=pl.BlockSpec((_TM, 128), lambda i: (i, 0)),
        out_shape=jax.ShapeDtypeStruct((_TBLROWS, 128), jnp.float32),
    )(embT, eye)


_NC = 2
_NS = 16
_NW = _NC * _NS
_B2 = 2 * BATCH
_BPW = _B2 // _NW
_CHUNK = 128
_STAGE = 256
_ROUNDS = _BPW // _STAGE


@functools.partial(
    pl.kernel,
    out_type=jax.ShapeDtypeStruct((2 * EMBED_DIM, BATCH), jnp.float32),
    mesh=plsc.VectorSubcoreMesh(
        core_axis_name="c", subcore_axis_name="s",
        num_cores=_NC, num_subcores=_NS),
    scratch_types=[
        pltpu.VMEM((_BPW,), jnp.int32),
        pltpu.VMEM((_BPW,), jnp.int32),
        pltpu.VMEM((_BPW,), jnp.int32),
        pltpu.VMEM((_STAGE, 128), jnp.float32),
        pltpu.VMEM((_STAGE, 128), jnp.float32),
        pltpu.VMEM((EMBED_DIM, _STAGE), jnp.float32),
        pltpu.VMEM((EMBED_DIM, _STAGE), jnp.float32),
        pltpu.SemaphoreType.DMA,
        pltpu.SemaphoreType.DMA,
    ],
    compiler_params=pltpu.CompilerParams(needs_layout_passes=False),
)
def _sc_gather(pk_hbm, table_hbm, out_hbm, idx_v, row_v, hcol_v,
               rows_a, rows_b, ext_a, ext_b, sem, wsem):
    wid = lax.axis_index("s") * _NC + lax.axis_index("c")
    base = wid * _BPW
    dbase = EMBED_DIM * (wid // 16)
    cbase = (wid % 16) * _BPW
    pltpu.sync_copy(pk_hbm.at[pl.ds(base, _BPW)], idx_v)
    for j in range(_BPW // 16):
        sl = pl.ds(j * 16, 16)
        pk = idx_v[sl]
        row_v[sl] = jnp.bitwise_and(pk, 0x1FFFF)
        hcol_v[sl] = jnp.left_shift(jnp.right_shift(pk, 17), 4)
    gathers, writes = {}, {}
    lane16 = lax.iota(jnp.int32, 16)

    def fire(r):
        buf = rows_a if r % 2 == 0 else rows_b
        return [pltpu.async_copy(
            table_hbm.at[row_v.at[pl.ds(r * _STAGE + j * _CHUNK, _CHUNK)]],
            buf.at[pl.ds(j * _CHUNK, _CHUNK)], sem)
            for j in range(_STAGE // _CHUNK)]

    def extract_and_write(r):
        for c in gathers[r]:
            c.wait()
        buf = rows_a if r % 2 == 0 else rows_b
        ext = ext_a if r % 2 == 0 else ext_b

        def body(k, _):
            rows16 = lane16 + k * 16
            hcol = hcol_v[pl.ds(r * _STAGE + k * 16, 16)]
            for d in range(EMBED_DIM):
                v = plsc.load_gather(buf, [rows16, hcol + d])
                plsc.store_scatter(
                    ext, [jnp.full((16,), d, jnp.int32), rows16], v)
            return 0

        lax.fori_loop(0, _STAGE // 16, body, 0)
        writes[r] = pltpu.async_copy(
            ext,
            out_hbm.at[pl.ds(dbase, EMBED_DIM),
                       pl.ds(cbase + r * _STAGE, _STAGE)], wsem)

    for r in range(_ROUNDS):
        if r - 2 in writes:
            writes[r - 2].wait()
        gathers[r] = fire(r)
        if r - 1 in gathers:
            extract_and_write(r - 1)
    extract_and_write(_ROUNDS - 1)
    writes[_ROUNDS - 2].wait()
    writes[_ROUNDS - 1].wait()


def _mlp_body(g_ref, w1_ref, b1_ref, w2_ref, b2_ref, o_ref):
    z = jax.lax.dot_general(w1_ref[...], g_ref[...], (((0,), (0,)), ((), ())),
                            preferred_element_type=jnp.float32)
    z = jnp.maximum(z + b1_ref[...], 0.0)
    o_ref[...] = (
        jax.lax.dot_general(w2_ref[...], z, (((0,), (0,)), ((), ())),
                            preferred_element_type=jnp.float32)
        + b2_ref[...])


_MLP_BLK = 2048


def _mlp(g, W1, b1, W2, b2):
    return pl.pallas_call(
        _mlp_body,
        grid=(BATCH // _MLP_BLK,),
        in_specs=[
            pl.BlockSpec((2 * EMBED_DIM, _MLP_BLK), lambda i: (0, i)),
            pl.BlockSpec((2 * EMBED_DIM, 64), lambda i: (0, 0)),
            pl.BlockSpec((64, 1), lambda i: (0, 0)),
            pl.BlockSpec((64, NUM_STAGES), lambda i: (0, 0)),
            pl.BlockSpec((NUM_STAGES, 1), lambda i: (0, 0)),
        ],
        out_specs=pl.BlockSpec((NUM_STAGES, _MLP_BLK), lambda i: (0, i)),
        out_shape=jax.ShapeDtypeStruct((NUM_STAGES, BATCH), jnp.float32),
    )(g, W1, b1.reshape(64, 1), W2, b2.reshape(NUM_STAGES, 1))


def kernel(x, emb, W1, b1, W2, b2):
    x = x.astype(jnp.int32)
    idx = jnp.concatenate([x[:, 0], x[:, 1]])
    rows = ((idx >> 16) << 13) + (idx & 8191)
    lane_grp = (idx >> 13) & 7
    packed = (lane_grp << 17) | rows
    table = _transpose(emb.T, jnp.eye(128, dtype=jnp.float32))
    g = _sc_gather(packed, table)
    out_t = _mlp(g, W1, b1, W2, b2)
    return out_t.T
